# baseline (speedup 1.0000x reference)
import functools

import jax
import jax.numpy as jnp
from jax import lax
from jax.experimental import pallas as pl
from jax.experimental.pallas import tpu as pltpu
from jax.experimental.pallas import tpu_sc as plsc

N = 320000
D = 128
S = 1024
NC = 2
NS = 16
NW = NC * NS
ROWS_PER_TILE = N // NW
CHUNK = 80
NCHUNK = ROWS_PER_TILE // CHUNK
NBUF = 5
M = 128
SEG_PER_TILE = S // NS


def _sc_segment_sum(x_hbm, batch_hbm, psum_hbm, xbuf, idxbuf, obuf, lacc,
                    fidx, acc, xsem, isem, ssem):
  cid = lax.axis_index("c")
  sid = lax.axis_index("s")
  wid = cid * NS + sid
  tile_base = wid * ROWS_PER_TILE
  iota = lax.iota(jnp.int32, 16)
  cols = [iota + 16 * k for k in range(D // 16)]
  zvec = jnp.zeros((16,), jnp.float32)

  def x_desc(c, b):
    return pltpu.make_async_copy(
        x_hbm.at[pl.ds(tile_base + c * CHUNK, CHUNK)], xbuf.at[b],
        xsem.at[b])

  def i_desc(c, b):
    return pltpu.make_async_copy(
        batch_hbm.at[pl.ds(tile_base + c * CHUNK, CHUNK)], idxbuf.at[b],
        isem.at[b])

  def s_desc(b):
    return pltpu.make_async_copy(xbuf.at[b], acc.at[idxbuf.at[b]], ssem.at[b])

  def lane_splat(v, j):
    return lax.gather(
        v, jnp.full((16, 1), j, jnp.int32),
        lax.GatherDimensionNumbers(
            offset_dims=(), collapsed_slice_dims=(0,), start_index_map=(0,)),
        (1,), mode=lax.GatherScatterMode.PROMISE_IN_BOUNDS)

  def zero_lacc():
    @pl.loop(0, M)
    def _(r):
      for k in range(D // 16):
        lacc[r, pl.ds(k * 16, 16)] = zvec

  def flush_window(wbase):
    for t in range(M // 16):
      fidx[pl.ds(t * 16, 16)] = iota + (wbase + 16 * t)
    pltpu.sync_copy(lacc, acc.at[fidx], add=True)
    zero_lacc()

  def fast_acc(b, wbase):
    for g in range(CHUNK // 16):
      rowv = idxbuf[b, pl.ds(g * 16, 16)] - wbase
      for j in range(16):
        ridv = lane_splat(rowv, j)
        for k in range(D // 16):
          data = xbuf[b, g * 16 + j, pl.ds(k * 16, 16)]
          plsc.addupdate_scatter(lacc, [ridv, cols[k]], data)

  def vec_chunk(b, base):
    ids_lo = idxbuf[b, pl.ds(0, 16)]
    ids_hi = idxbuf[b, pl.ds(CHUNK - 16, 16)]
    m0 = lane_splat(ids_lo, 0)
    mx = lane_splat(ids_hi, 15)
    missv = jnp.logical_or(base < 0, mx >= base + M)
    fitsv = (mx - m0) < M
    miss = jnp.any(missv)
    fits = jnp.any(fitsv)
    do_flush = jnp.logical_and(miss, jnp.any(base >= 0))
    new_base = jnp.where(missv, jnp.where(fitsv, m0, jnp.int32(-1)), base)

    @pl.when(do_flush)
    def _():
      flush_window(base)

    @pl.when(jnp.logical_or(jnp.logical_not(miss), fits))
    def _():
      fast_acc(b, new_base)

    @pl.when(jnp.logical_and(miss, jnp.logical_not(fits)))
    def _():
      pltpu.sync_copy(xbuf.at[b], acc.at[idxbuf.at[b]], add=True)

    return new_base

  @pl.loop(0, SEG_PER_TILE)
  def _(i):
    for j in range(D // 16):
      obuf[i, pl.ds(j * 16, 16)] = zvec

  pltpu.sync_copy(obuf, acc.at[pl.ds(sid * SEG_PER_TILE, SEG_PER_TILE)])
  zero_lacc()
  plsc.subcore_barrier()

  for b in range(NBUF - 1):
    x_desc(b, b).start()
    i_desc(b, b).start()

  @pl.loop(0, NCHUNK, step=NBUF,
           init_carry=jnp.full((16,), -1, jnp.int32))
  def base(i, base):
    for b in range(NBUF):
      c = i + b
      nxt = c + NBUF - 1
      nb = (b + NBUF - 1) % NBUF

      @pl.when(nxt < NCHUNK)
      def _():
        if nb == 0:
          s_desc(nb).wait()
        x_desc(nxt, nb).start()
        i_desc(nxt, nb).start()

      x_desc(c, b).wait()
      i_desc(c, b).wait()
      if b == 0:
        pltpu.async_copy(xbuf.at[b], acc.at[idxbuf.at[b]], ssem.at[b],
                         add=True)
      else:
        base = vec_chunk(b, base)
    return base

  @pl.when(jnp.any(base >= 0))
  def _():
    flush_window(base)

  s_desc(0).wait()
  plsc.subcore_barrier()

  seg0 = sid * SEG_PER_TILE
  pltpu.sync_copy(acc.at[pl.ds(seg0, SEG_PER_TILE)], obuf)
  pltpu.sync_copy(obuf, psum_hbm.at[cid, pl.ds(seg0, SEG_PER_TILE)])


_sc_call = functools.partial(
    pl.kernel,
    out_type=jax.ShapeDtypeStruct((NC, S, D), jnp.float32),
    mesh=plsc.VectorSubcoreMesh(core_axis_name="c", subcore_axis_name="s"),
    scratch_types=[
        pltpu.VMEM((NBUF, CHUNK, D), jnp.float32),
        pltpu.VMEM((NBUF, CHUNK), jnp.int32),
        pltpu.VMEM((SEG_PER_TILE, D), jnp.float32),
        pltpu.VMEM((M, D), jnp.float32),
        pltpu.VMEM((M,), jnp.int32),
        pltpu.VMEM_SHARED((S, D), jnp.float32),
        pltpu.SemaphoreType.DMA((NBUF,)),
        pltpu.SemaphoreType.DMA((NBUF,)),
        pltpu.SemaphoreType.DMA((NBUF,)),
    ],
    compiler_params=pltpu.CompilerParams(needs_layout_passes=False),
)(_sc_segment_sum)


def _mm_body(psum_ref, w_ref, b_ref, out_ref):
  pooled = psum_ref[0] + psum_ref[1]
  out_ref[...] = lax.dot_general(
      pooled, w_ref[...], (((1,), (1,)), ((), ())),
      preferred_element_type=jnp.float32) + b_ref[...]


_mm_call = pl.pallas_call(
    _mm_body,
    out_shape=jax.ShapeDtypeStruct((S, D), jnp.float32),
)


def kernel(x, batch, W, b):
  psum = _sc_call(x, batch.astype(jnp.int32))
  return _mm_call(psum, W, b.reshape(1, D))


# speedup vs baseline: 1.3032x; 1.3032x over previous
import functools

import jax
import jax.numpy as jnp
from jax import lax
from jax.experimental import pallas as pl
from jax.experimental.pallas import tpu as pltpu
from jax.experimental.pallas import tpu_sc as plsc

N = 320000
D = 128
S = 1024
NC = 2
NS = 16
NW = NC * NS
ROWS_PER_TILE = N // NW
CHUNK = 80
NCHUNK = ROWS_PER_TILE // CHUNK
NBUF = 5
M = 128
SEG_PER_TILE = S // NS


def _sc_segment_sum(x_hbm, batch_hbm, psum_hbm, xbuf, idxbuf, obuf, lacc,
                    fidx, acc, xsem, isem, ssem):
  cid = lax.axis_index("c")
  sid = lax.axis_index("s")
  wid = cid * NS + sid
  tile_base = wid * ROWS_PER_TILE
  iota = lax.iota(jnp.int32, 16)
  cols = [iota + 16 * k for k in range(D // 16)]
  zvec = jnp.zeros((16,), jnp.float32)

  def x_desc(c, b):
    return pltpu.make_async_copy(
        x_hbm.at[pl.ds(tile_base + c * CHUNK, CHUNK)], xbuf.at[b],
        xsem.at[b])

  def i_desc(c, b):
    return pltpu.make_async_copy(
        batch_hbm.at[pl.ds(tile_base + c * CHUNK, CHUNK)], idxbuf.at[b],
        isem.at[b])

  def s_desc(b):
    return pltpu.make_async_copy(xbuf.at[b], acc.at[idxbuf.at[b]], ssem.at[b])

  def lane_splat(v, j):
    return lax.gather(
        v, jnp.full((16, 1), j, jnp.int32),
        lax.GatherDimensionNumbers(
            offset_dims=(), collapsed_slice_dims=(0,), start_index_map=(0,)),
        (1,), mode=lax.GatherScatterMode.PROMISE_IN_BOUNDS)

  def zero_lacc():
    @pl.loop(0, M)
    def _(r):
      for k in range(D // 16):
        lacc[r, pl.ds(k * 16, 16)] = zvec

  def flush_window(wbase):
    for t in range(M // 16):
      fidx[pl.ds(t * 16, 16)] = iota + (wbase + 16 * t)
    pltpu.sync_copy(lacc, acc.at[fidx], add=True)
    zero_lacc()

  def fast_acc(b, wbase):
    @plsc.parallel_loop(0, CHUNK // 16)
    def _(g):
      rowv = idxbuf[b, pl.ds(g * 16, 16)] - wbase
      for j in range(16):
        ridv = lane_splat(rowv, j)
        for k in range(D // 16):
          data = xbuf[b, g * 16 + j, pl.ds(k * 16, 16)]
          plsc.addupdate_scatter(lacc, [ridv, cols[k]], data)

  def vec_chunk(b, base):
    ids_lo = idxbuf[b, pl.ds(0, 16)]
    ids_hi = idxbuf[b, pl.ds(CHUNK - 16, 16)]
    m0 = lane_splat(ids_lo, 0)
    mx = lane_splat(ids_hi, 15)
    missv = jnp.logical_or(base < 0, mx >= base + M)
    fitsv = (mx - m0) < M
    miss = jnp.any(missv)
    fits = jnp.any(fitsv)
    do_flush = jnp.logical_and(miss, jnp.any(base >= 0))
    new_base = jnp.where(missv, jnp.where(fitsv, m0, jnp.int32(-1)), base)

    @pl.when(do_flush)
    def _():
      flush_window(base)

    @pl.when(jnp.logical_or(jnp.logical_not(miss), fits))
    def _():
      fast_acc(b, new_base)

    @pl.when(jnp.logical_and(miss, jnp.logical_not(fits)))
    def _():
      pltpu.sync_copy(xbuf.at[b], acc.at[idxbuf.at[b]], add=True)

    return new_base

  @pl.loop(0, SEG_PER_TILE)
  def _(i):
    for j in range(D // 16):
      obuf[i, pl.ds(j * 16, 16)] = zvec

  pltpu.sync_copy(obuf, acc.at[pl.ds(sid * SEG_PER_TILE, SEG_PER_TILE)])
  zero_lacc()
  plsc.subcore_barrier()

  for b in range(NBUF - 1):
    x_desc(b, b).start()
    i_desc(b, b).start()

  @pl.loop(0, NCHUNK, step=NBUF,
           init_carry=jnp.full((16,), -1, jnp.int32))
  def base(i, base):
    for b in range(NBUF):
      c = i + b
      nxt = c + NBUF - 1
      nb = (b + NBUF - 1) % NBUF

      @pl.when(nxt < NCHUNK)
      def _():
        if nb == 0:
          s_desc(nb).wait()
        x_desc(nxt, nb).start()
        i_desc(nxt, nb).start()

      x_desc(c, b).wait()
      i_desc(c, b).wait()
      if b == 0:
        pltpu.async_copy(xbuf.at[b], acc.at[idxbuf.at[b]], ssem.at[b],
                         add=True)
      else:
        base = vec_chunk(b, base)
    return base

  @pl.when(jnp.any(base >= 0))
  def _():
    flush_window(base)

  s_desc(0).wait()
  plsc.subcore_barrier()

  seg0 = sid * SEG_PER_TILE
  pltpu.sync_copy(acc.at[pl.ds(seg0, SEG_PER_TILE)], obuf)
  pltpu.sync_copy(obuf, psum_hbm.at[cid, pl.ds(seg0, SEG_PER_TILE)])


_sc_call = functools.partial(
    pl.kernel,
    out_type=jax.ShapeDtypeStruct((NC, S, D), jnp.float32),
    mesh=plsc.VectorSubcoreMesh(core_axis_name="c", subcore_axis_name="s"),
    scratch_types=[
        pltpu.VMEM((NBUF, CHUNK, D), jnp.float32),
        pltpu.VMEM((NBUF, CHUNK), jnp.int32),
        pltpu.VMEM((SEG_PER_TILE, D), jnp.float32),
        pltpu.VMEM((M, D), jnp.float32),
        pltpu.VMEM((M,), jnp.int32),
        pltpu.VMEM_SHARED((S, D), jnp.float32),
        pltpu.SemaphoreType.DMA((NBUF,)),
        pltpu.SemaphoreType.DMA((NBUF,)),
        pltpu.SemaphoreType.DMA((NBUF,)),
    ],
    compiler_params=pltpu.CompilerParams(needs_layout_passes=False),
)(_sc_segment_sum)


def _mm_body(psum_ref, w_ref, b_ref, out_ref):
  pooled = psum_ref[0] + psum_ref[1]
  out_ref[...] = lax.dot_general(
      pooled, w_ref[...], (((1,), (1,)), ((), ())),
      preferred_element_type=jnp.float32) + b_ref[...]


_mm_call = pl.pallas_call(
    _mm_body,
    out_shape=jax.ShapeDtypeStruct((S, D), jnp.float32),
)


def kernel(x, batch, W, b):
  psum = _sc_call(x, batch.astype(jnp.int32))
  return _mm_call(psum, W, b.reshape(1, D))


# speedup vs baseline: 2.6573x; 2.0391x over previous
import functools

import jax
import jax.numpy as jnp
from jax import lax
from jax.experimental import pallas as pl
from jax.experimental.pallas import tpu as pltpu
from jax.experimental.pallas import tpu_sc as plsc

N = 320000
D = 128
S = 1024
NC = 2
NS = 16
NW = NC * NS
ROWS_PER_TILE = N // NW
CHUNK = 80
NCHUNK = ROWS_PER_TILE // CHUNK
NBUF = 5
SEG_PER_TILE = S // NS


def _sc_segment_sum(x_hbm, batch_hbm, psum_hbm, xbuf, idxbuf, obuf, acc,
                    xsem, isem, ssem):
  cid = lax.axis_index("c")
  sid = lax.axis_index("s")
  wid = cid * NS + sid
  tile_base = wid * ROWS_PER_TILE

  def x_desc(c, b):
    return pltpu.make_async_copy(
        x_hbm.at[pl.ds(tile_base + c * CHUNK, CHUNK)], xbuf.at[b],
        xsem.at[b])

  def i_desc(c, b):
    return pltpu.make_async_copy(
        batch_hbm.at[pl.ds(tile_base + c * CHUNK, CHUNK)], idxbuf.at[b],
        isem.at[b])

  def s_desc(b):
    return pltpu.make_async_copy(xbuf.at[b], acc.at[idxbuf.at[b]], ssem.at[b])

  @pl.loop(0, SEG_PER_TILE)
  def _(i):
    for j in range(D // 16):
      obuf[i, pl.ds(j * 16, 16)] = jnp.zeros((16,), jnp.float32)

  pltpu.sync_copy(obuf, acc.at[pl.ds(sid * SEG_PER_TILE, SEG_PER_TILE)])
  plsc.subcore_barrier()

  for b in range(NBUF - 1):
    x_desc(b, b).start()
    i_desc(b, b).start()

  @pl.loop(0, NCHUNK, step=NBUF)
  def _(i):
    for b in range(NBUF):
      c = i + b
      nxt = c + NBUF - 1
      nb = (b + NBUF - 1) % NBUF

      @pl.when(nxt < NCHUNK)
      def _():
        if b == 0:
          @pl.when(c >= 1)
          def _():
            s_desc(nb).wait()
        else:
          s_desc(nb).wait()
        x_desc(nxt, nb).start()
        i_desc(nxt, nb).start()

      x_desc(c, b).wait()
      i_desc(c, b).wait()
      pltpu.async_copy(xbuf.at[b], acc.at[idxbuf.at[b]], ssem.at[b],
                       add=True, priority=1)

  for b in range(NBUF):
    s_desc(b).wait()

  plsc.subcore_barrier()

  seg0 = sid * SEG_PER_TILE
  pltpu.sync_copy(acc.at[pl.ds(seg0, SEG_PER_TILE)], obuf)
  pltpu.sync_copy(obuf, psum_hbm.at[cid, pl.ds(seg0, SEG_PER_TILE)])


_sc_call = functools.partial(
    pl.kernel,
    out_type=jax.ShapeDtypeStruct((NC, S, D), jnp.float32),
    mesh=plsc.VectorSubcoreMesh(core_axis_name="c", subcore_axis_name="s"),
    scratch_types=[
        pltpu.VMEM((NBUF, CHUNK, D), jnp.float32),
        pltpu.VMEM((NBUF, CHUNK), jnp.int32),
        pltpu.VMEM((SEG_PER_TILE, D), jnp.float32),
        pltpu.VMEM_SHARED((S, D), jnp.float32),
        pltpu.SemaphoreType.DMA((NBUF,)),
        pltpu.SemaphoreType.DMA((NBUF,)),
        pltpu.SemaphoreType.DMA((NBUF,)),
    ],
)(_sc_segment_sum)


def _mm_body(psum_ref, w_ref, b_ref, out_ref):
  pooled = psum_ref[0] + psum_ref[1]
  out_ref[...] = lax.dot_general(
      pooled, w_ref[...], (((1,), (1,)), ((), ())),
      preferred_element_type=jnp.float32) + b_ref[...]


_mm_call = pl.pallas_call(
    _mm_body,
    out_shape=jax.ShapeDtypeStruct((S, D), jnp.float32),
)


def kernel(x, batch, W, b):
  psum = _sc_call(x, batch.astype(jnp.int32))
  return _mm_call(psum, W, b.reshape(1, D))


# speedup vs baseline: 2.7792x; 1.0459x over previous
import functools

import jax
import jax.numpy as jnp
from jax import lax
from jax.experimental import pallas as pl
from jax.experimental.pallas import tpu as pltpu
from jax.experimental.pallas import tpu_sc as plsc

N = 320000
D = 128
S = 1024
NC = 2
NS = 16
NW = NC * NS
ROWS_PER_TILE = N // NW
CHUNK = 128
NCHUNK = ROWS_PER_TILE // CHUNK
TAIL = ROWS_PER_TILE - NCHUNK * CHUNK
NBUF = 6
SEG_PER_TILE = S // NS


def _sc_segment_sum(x_hbm, batch_hbm, psum_hbm, xbuf, idxbuf, tidx, obuf,
                    acc, xsem, isem, ssem):
  cid = lax.axis_index("c")
  sid = lax.axis_index("s")
  wid = cid * NS + sid
  tile_base = wid * ROWS_PER_TILE

  def x_desc(c, b):
    return pltpu.make_async_copy(
        x_hbm.at[pl.ds(tile_base + c * CHUNK, CHUNK)], xbuf.at[b],
        xsem.at[b])

  def i_desc(c, b):
    return pltpu.make_async_copy(
        batch_hbm.at[pl.ds(tile_base + c * CHUNK, CHUNK)], idxbuf.at[b],
        isem.at[b])

  def s_desc(b):
    return pltpu.make_async_copy(xbuf.at[b], acc.at[idxbuf.at[b]], ssem.at[b])

  @pl.loop(0, SEG_PER_TILE)
  def _(i):
    for j in range(D // 16):
      obuf[i, pl.ds(j * 16, 16)] = jnp.zeros((16,), jnp.float32)

  pltpu.sync_copy(obuf, acc.at[pl.ds(sid * SEG_PER_TILE, SEG_PER_TILE)])
  plsc.subcore_barrier()

  for b in range(NBUF - 1):
    x_desc(b, b).start()
    i_desc(b, b).start()

  @pl.loop(0, NCHUNK, step=NBUF)
  def _(i):
    for b in range(NBUF):
      c = i + b
      nxt = c + NBUF - 1
      nb = (b + NBUF - 1) % NBUF

      @pl.when(nxt < NCHUNK)
      def _():
        if b == 0:
          @pl.when(c >= 1)
          def _():
            s_desc(nb).wait()
        else:
          s_desc(nb).wait()
        x_desc(nxt, nb).start()
        i_desc(nxt, nb).start()

      x_desc(c, b).wait()
      i_desc(c, b).wait()
      pltpu.async_copy(xbuf.at[b], acc.at[idxbuf.at[b]], ssem.at[b],
                       add=True, priority=1)

  for b in range(NBUF):
    s_desc(b).wait()

  tail_base = tile_base + NCHUNK * CHUNK
  pltpu.sync_copy(x_hbm.at[pl.ds(tail_base, TAIL)],
                  xbuf.at[0, pl.ds(0, TAIL)])
  pltpu.sync_copy(batch_hbm.at[pl.ds(tail_base, TAIL)], tidx.at[0])
  pltpu.sync_copy(xbuf.at[0, pl.ds(0, TAIL)], acc.at[tidx.at[0]], add=True)

  plsc.subcore_barrier()

  seg0 = sid * SEG_PER_TILE
  pltpu.sync_copy(acc.at[pl.ds(seg0, SEG_PER_TILE)], obuf)
  pltpu.sync_copy(obuf, psum_hbm.at[cid, pl.ds(seg0, SEG_PER_TILE)])


_sc_call = functools.partial(
    pl.kernel,
    out_type=jax.ShapeDtypeStruct((NC, S, D), jnp.float32),
    mesh=plsc.VectorSubcoreMesh(core_axis_name="c", subcore_axis_name="s"),
    scratch_types=[
        pltpu.VMEM((NBUF, CHUNK, D), jnp.float32),
        pltpu.VMEM((NBUF, CHUNK), jnp.int32),
        pltpu.VMEM((1, TAIL), jnp.int32),
        pltpu.VMEM((SEG_PER_TILE, D), jnp.float32),
        pltpu.VMEM_SHARED((S, D), jnp.float32),
        pltpu.SemaphoreType.DMA((NBUF,)),
        pltpu.SemaphoreType.DMA((NBUF,)),
        pltpu.SemaphoreType.DMA((NBUF,)),
    ],
)(_sc_segment_sum)


def _mm_body(psum_ref, w_ref, b_ref, out_ref):
  pooled = psum_ref[0] + psum_ref[1]
  out_ref[...] = lax.dot_general(
      pooled, w_ref[...], (((1,), (1,)), ((), ())),
      preferred_element_type=jnp.float32) + b_ref[...]


_mm_call = pl.pallas_call(
    _mm_body,
    out_shape=jax.ShapeDtypeStruct((S, D), jnp.float32),
)


def kernel(x, batch, W, b):
  psum = _sc_call(x, batch.astype(jnp.int32))
  return _mm_call(psum, W, b.reshape(1, D))


# speedup vs baseline: 3.1149x; 1.1208x over previous
import functools

import jax
import jax.numpy as jnp
from jax import lax
from jax.experimental import pallas as pl
from jax.experimental.pallas import tpu as pltpu
from jax.experimental.pallas import tpu_sc as plsc

N = 320000
D = 128
S = 1024
NC = 2
NS = 16
NW = NC * NS
SB = 128
LCHUNK = 256
RPT = 10240
RPT_LAST = N - (NW - 1) * RPT
NCH_FULL = RPT // LCHUNK
NCH_LAST = RPT_LAST // LCHUNK
IDR = RPT // SB
IDROWS_PAD = NW * IDR
NBUF = 2
SEG_PER_TILE = S // NS


def _sc_segment_sum(x_hbm, ids_hbm, psum_hbm, xbuf, idxbuf, obuf, acc,
                    xsem, ssem):
  cid = lax.axis_index("c")
  sid = lax.axis_index("s")
  wid = cid * NS + sid
  tile_base = wid * RPT
  nch = jnp.where(wid == NW - 1, NCH_LAST, NCH_FULL)

  def x_desc(c, b):
    return pltpu.make_async_copy(
        x_hbm.at[pl.ds(tile_base + c * LCHUNK, LCHUNK)], xbuf.at[b],
        xsem.at[b])

  def s_descs(c, b):
    return [
        pltpu.make_async_copy(
            xbuf.at[b, pl.ds(j * SB, SB)],
            acc.at[idxbuf.at[c * (LCHUNK // SB) + j]], ssem.at[b])
        for j in range(LCHUNK // SB)
    ]

  pltpu.sync_copy(ids_hbm.at[pl.ds(wid * IDR, IDR)], idxbuf)

  @pl.loop(0, SEG_PER_TILE)
  def _(i):
    for j in range(D // 16):
      obuf[i, pl.ds(j * 16, 16)] = jnp.zeros((16,), jnp.float32)

  pltpu.sync_copy(obuf, acc.at[pl.ds(sid * SEG_PER_TILE, SEG_PER_TILE)])
  plsc.subcore_barrier()

  x_desc(0, 0).start()

  @pl.loop(0, nch, step=NBUF)
  def _(i):
    for b in range(NBUF):
      c = i + b
      nxt = c + 1
      nb = (b + 1) % NBUF

      @pl.when(c < nch)
      def _():
        @pl.when(nxt < nch)
        def _():
          @pl.when(c >= 1)
          def _():
            for d in s_descs(c - 1, nb):
              d.wait()
          x_desc(nxt, nb).start()

        x_desc(c, b).wait()
        for j in range(LCHUNK // SB):
          pltpu.async_copy(
              xbuf.at[b, pl.ds(j * SB, SB)],
              acc.at[idxbuf.at[c * (LCHUNK // SB) + j]], ssem.at[b],
              add=True)

  for d in s_descs(nch - 2, 0):
    d.wait()
  for d in s_descs(nch - 1, 1):
    d.wait()

  plsc.subcore_barrier()

  seg0 = sid * SEG_PER_TILE
  pltpu.sync_copy(acc.at[pl.ds(seg0, SEG_PER_TILE)], obuf)
  pltpu.sync_copy(obuf, psum_hbm.at[cid, pl.ds(seg0, SEG_PER_TILE)])


_sc_call = functools.partial(
    pl.kernel,
    out_type=jax.ShapeDtypeStruct((NC, S, D), jnp.float32),
    mesh=plsc.VectorSubcoreMesh(core_axis_name="c", subcore_axis_name="s"),
    scratch_types=[
        pltpu.VMEM((NBUF, LCHUNK, D), jnp.float32),
        pltpu.VMEM((IDR, SB), jnp.int32),
        pltpu.VMEM((SEG_PER_TILE, D), jnp.float32),
        pltpu.VMEM_SHARED((S, D), jnp.float32),
        pltpu.SemaphoreType.DMA((NBUF,)),
        pltpu.SemaphoreType.DMA((NBUF,)),
    ],
)(_sc_segment_sum)


def _mm_body(psum_ref, w_ref, b_ref, out_ref):
  pooled = psum_ref[0] + psum_ref[1]
  out_ref[...] = lax.dot_general(
      pooled, w_ref[...], (((1,), (1,)), ((), ())),
      preferred_element_type=jnp.float32) + b_ref[...]


_mm_call = pl.pallas_call(
    _mm_body,
    out_shape=jax.ShapeDtypeStruct((S, D), jnp.float32),
)


def kernel(x, batch, W, b):
  ids = batch.astype(jnp.int32).reshape(N // SB, SB)
  ids = jnp.pad(ids, ((0, IDROWS_PAD - N // SB), (0, 0)))
  psum = _sc_call(x, ids)
  return _mm_call(psum, W, b.reshape(1, D))
